# Initial kernel scaffold; baseline (speedup 1.0000x reference)
import jax
import jax.numpy as jnp
from jax.experimental import pallas as pl


def kernel(predicted_locs, predicted_scores, boxes, labels, priors_cxcy):
    raise NotImplementedError("write your pallas kernel here")


# speedup vs baseline: 22.9799x; 22.9799x over previous
import jax
import jax.numpy as jnp
from jax.experimental import pallas as pl
from jax.experimental.pallas import tpu as pltpu

_B, _C, _P, _X = 8, 8, 8732, 16
_THRESHOLD = 0.5
_NEG_POS_RATIO = 3
_ALPHA = 1.0


def _body(boxes_ref, labels_ref, priors_ref, pl_ref, ps_ref, out_ref, acc_ref):
    pid = pl.program_id(0)

    @pl.when(pid == 0)
    def _init():
        for j in range(4):
            for c in range(_C):
                acc_ref[j, c] = 0.0
        out_ref[0, 0] = 0.0

    boxes = boxes_ref[0]
    labf = labels_ref[0]
    pri = priors_ref[...]
    plc = pl_ref[0]
    ps = ps_ref[0]

    pcx = pri[0:1, :]
    pcy = pri[1:2, :]
    pw = pri[2:3, :]
    ph = pri[3:4, :]
    px0 = pcx - pw * 0.5
    py0 = pcy - ph * 0.5
    px1 = pcx + pw * 0.5
    py1 = pcy + ph * 0.5

    bx0 = boxes[:, 0:1]
    by0 = boxes[:, 1:2]
    bx1 = boxes[:, 2:3]
    by1 = boxes[:, 3:4]

    wx = jnp.maximum(jnp.minimum(bx1, px1) - jnp.maximum(bx0, px0), 0.0)
    wy = jnp.maximum(jnp.minimum(by1, py1) - jnp.maximum(by0, py0), 0.0)
    inter = wx * wy
    area_a = (bx1 - bx0) * (by1 - by0)
    area_b = (px1 - px0) * (py1 - py0)
    iou = inter / (area_a + area_b - inter)

    row_iota = jax.lax.broadcasted_iota(jnp.int32, (_X, _P), 0)
    col_iota = jax.lax.broadcasted_iota(jnp.int32, (_X, _P), 1)

    maxv = jnp.max(iou, axis=0, keepdims=True)
    oidx0 = jnp.min(jnp.where(iou == maxv, row_iota, _X),
                    axis=0, keepdims=True)
    mrow = jnp.max(iou, axis=1, keepdims=True)
    pfo = jnp.min(jnp.where(iou == mrow, col_iota, _P),
                  axis=1, keepdims=True)

    fmask = col_iota == pfo
    forced = jnp.max(jnp.where(fmask, 1, 0), axis=0, keepdims=True)
    fidx = jnp.max(jnp.where(fmask, row_iota, -1), axis=0, keepdims=True)
    oidx = jnp.where(forced == 1, fidx, oidx0)
    ofp = jnp.where(forced == 1, 1.0, maxv)

    onehot = row_iota == oidx
    g_x0 = jnp.sum(jnp.where(onehot, bx0, 0.0), axis=0, keepdims=True)
    g_y0 = jnp.sum(jnp.where(onehot, by0, 0.0), axis=0, keepdims=True)
    g_x1 = jnp.sum(jnp.where(onehot, bx1, 0.0), axis=0, keepdims=True)
    g_y1 = jnp.sum(jnp.where(onehot, by1, 0.0), axis=0, keepdims=True)
    labcol = jnp.transpose(labf)
    g_lab = jnp.sum(jnp.where(onehot, labcol, 0.0), axis=0, keepdims=True)

    pos = (ofp >= _THRESHOLD) & (g_lab != 0.0)

    cx = (g_x0 + g_x1) * 0.5
    cy = (g_y0 + g_y1) * 0.5
    w = g_x1 - g_x0
    h = g_y1 - g_y0
    t0 = (cx - pcx) / (pw * 0.1)
    t1 = (cy - pcy) / (ph * 0.1)
    t2 = jnp.log(w / pw) * 5.0
    t3 = jnp.log(h / ph) * 5.0

    la = (jnp.abs(plc[0:1, :] - t0) + jnp.abs(plc[1:2, :] - t1)
          + jnp.abs(plc[2:3, :] - t2) + jnp.abs(plc[3:4, :] - t3))
    loc_sum = jnp.sum(jnp.where(pos, la, 0.0))

    s0 = ps[0:1, :]
    s1 = ps[1:2, :]
    mx = jnp.maximum(s0, s1)
    lse = mx + jnp.log(jnp.exp(s0 - mx) + jnp.exp(s1 - mx))
    ce = lse - jnp.where(pos, s1, s0)

    npos_i = jnp.sum(pos.astype(jnp.int32))
    pos_ce = jnp.sum(jnp.where(pos, ce, 0.0))
    ceneg = jnp.where(pos, 0.0, ce)

    k_eff = jnp.minimum(_NEG_POS_RATIO * npos_i, _P)
    vbits = jax.lax.bitcast_convert_type(ceneg, jnp.int32)

    def bit_step(i, pref):
        t = jnp.bitwise_or(pref, jnp.left_shift(jnp.int32(1), 30 - i))
        cnt = jnp.sum((vbits >= t).astype(jnp.int32))
        return jnp.where(cnt >= k_eff, t, pref)

    tbits = jax.lax.fori_loop(0, 31, bit_step, jnp.int32(0))
    gt = vbits > tbits
    sum_gt = jnp.sum(jnp.where(gt, ceneg, 0.0))
    cnt_gt = jnp.sum(gt.astype(jnp.int32))
    tval = jnp.max(jnp.where(vbits == tbits, ceneg, 0.0))
    neg_ce = sum_gt + (k_eff - cnt_gt).astype(jnp.float32) * tval

    c = pid % _C
    acc_ref[0, c] += npos_i.astype(jnp.float32)
    acc_ref[1, c] += loc_sum
    acc_ref[2, c] += pos_ce
    acc_ref[3, c] += neg_ce

    @pl.when(pid == pl.num_programs(0) - 1)
    def _final():
        total = 0.0
        for cc in range(_C):
            npos = acc_ref[0, cc]
            nps = jnp.maximum(npos, 1.0)
            loc_l = acc_ref[1, cc] / (nps * 4.0)
            conf = acc_ref[2, cc] + acc_ref[3, cc]
            contrib = (1.0 / _C) * (conf + _ALPHA * loc_l) / nps
            total += jnp.where(npos == 0.0, 0.0, contrib)
        out_ref[0, 0] = total


def kernel(predicted_locs, predicted_scores, boxes, labels, priors_cxcy):
    bc = _B * _C
    pl_t = predicted_locs.reshape(bc, _P, 4).transpose(0, 2, 1)
    ps_t = predicted_scores.reshape(bc, _P, 2).transpose(0, 2, 1)
    boxes_r = boxes.reshape(bc, _X, 4)
    labels_r = labels.astype(jnp.float32).reshape(bc, 1, _X)
    priors_t = priors_cxcy.transpose(1, 0)

    out = pl.pallas_call(
        _body,
        grid=(bc,),
        in_specs=[
            pl.BlockSpec((1, _X, 4), lambda p: (p, 0, 0)),
            pl.BlockSpec((1, 1, _X), lambda p: (p, 0, 0)),
            pl.BlockSpec((4, _P), lambda p: (0, 0)),
            pl.BlockSpec((1, 4, _P), lambda p: (p, 0, 0)),
            pl.BlockSpec((1, 2, _P), lambda p: (p, 0, 0)),
        ],
        out_specs=pl.BlockSpec(memory_space=pltpu.SMEM),
        out_shape=jax.ShapeDtypeStruct((1, 1), jnp.float32),
        scratch_shapes=[pltpu.SMEM((4, _C), jnp.float32)],
    )(boxes_r, labels_r, priors_t, pl_t, ps_t)
    return out[0, 0]
